# Initial kernel scaffold; baseline (speedup 1.0000x reference)
import jax
import jax.numpy as jnp
from jax.experimental import pallas as pl


def kernel(pred, target):
    raise NotImplementedError("write your pallas kernel here")


# speedup vs baseline: 2.2803x; 2.2803x over previous
import functools

import jax
import jax.numpy as jnp
from jax.experimental import pallas as pl
from jax.experimental.pallas import tpu as pltpu

_SMOOTHING = 0.1
_BLK_C = 2048


def _loss_kernel(pred_ref, target_ref, out_ref, *, n_classes, blk_c, eps, conf):
    j = pl.program_id(0)

    @pl.when(j == 0)
    def _():
        out_ref[0, 0] = jnp.float32(0.0)

    x = pred_ref[...]
    cols = j * blk_c + jax.lax.broadcasted_iota(jnp.int32, x.shape, 1)
    t = target_ref[...]
    w = jnp.where(cols == t, jnp.float32(conf), jnp.float32(eps))
    valid = cols < n_classes
    contrib = jnp.sum(jnp.where(valid, x * w, jnp.float32(0.0)))
    out_ref[0, 0] += contrib


@jax.jit
def kernel(pred, target):
    b, c = pred.shape
    eps = _SMOOTHING / (c - 1)
    conf = 1.0 - _SMOOTHING
    n_blocks = pl.cdiv(c, _BLK_C)
    t2d = target.astype(jnp.int32).reshape(b, 1)

    acc = pl.pallas_call(
        functools.partial(
            _loss_kernel, n_classes=c, blk_c=_BLK_C, eps=eps, conf=conf
        ),
        grid=(n_blocks,),
        in_specs=[
            pl.BlockSpec((b, _BLK_C), lambda j: (0, j)),
            pl.BlockSpec((b, 1), lambda j: (0, 0)),
        ],
        out_specs=pl.BlockSpec(
            (1, 1), lambda j: (0, 0), memory_space=pltpu.SMEM
        ),
        out_shape=jax.ShapeDtypeStruct((1, 1), jnp.float32),
        compiler_params=pltpu.CompilerParams(
            dimension_semantics=("arbitrary",),
        ),
    )(pred, t2d)

    return -acc[0, 0] / b
